# baseline (speedup 1.0000x reference)
import jax
import jax.numpy as jnp
from jax.experimental import pallas as pl
from jax.experimental.pallas import tpu as pltpu

_Q = 75
_C = 5
_SHOTS = 5
_D = 384
_L = 196
_S = _SHOTS * _L
_B = 5
_G = _Q // _B


def _norm_support_kernel(s_ref, out_ref):
    s = s_ref[0]
    ssq = jnp.sum(s * s, axis=0, keepdims=True)
    out_ref[0] = (s * jax.lax.rsqrt(ssq)).astype(jnp.bfloat16)


def _img2class_kernel(q_ref, sn_ref, out_ref):
    neg = jnp.float32(-jnp.inf)
    q = q_ref[0]
    qn = (q * jax.lax.rsqrt(jnp.sum(q * q, axis=0, keepdims=True))).astype(jnp.bfloat16)
    sims = [[None] * _C for _ in range(_B)]
    for c in range(_C):
        sc = sn_ref[c]
        x = jax.lax.dot_general(
            qn, sc, (((0,), (0,)), ((), ())),
            preferred_element_type=jnp.float32,
            precision=jax.lax.Precision.DEFAULT,
        )
        m1 = jnp.max(x, axis=1, keepdims=True)
        x2 = jnp.where(x == m1, neg, x)
        m2 = jnp.max(x2, axis=1, keepdims=True)
        x3 = jnp.where(x2 == m2, neg, x2)
        m3 = jnp.max(x3, axis=1, keepdims=True)
        m = m1 + m2 + m3
        for b in range(_B):
            sims[b][c] = jnp.sum(m[b * _L:(b + 1) * _L], axis=(0, 1),
                                 keepdims=True)
    for b in range(_B):
        out_ref[b] = jnp.concatenate(sims[b], axis=1)


def kernel(x1, x2):
    q = (x1.reshape(_G, _B, _D, _L)
         .transpose(0, 2, 1, 3)
         .reshape(_G, _D, _B * _L))
    s = x2.reshape(_C, _SHOTS, _D, _L).transpose(0, 2, 1, 3).reshape(_C, _D, _S)
    sn = pl.pallas_call(
        _norm_support_kernel,
        grid=(_C,),
        in_specs=[pl.BlockSpec((1, _D, _S), lambda c: (c, 0, 0))],
        out_specs=pl.BlockSpec((1, _D, _S), lambda c: (c, 0, 0)),
        out_shape=jax.ShapeDtypeStruct((_C, _D, _S), jnp.bfloat16),
    )(s)
    out = pl.pallas_call(
        _img2class_kernel,
        grid=(_G,),
        in_specs=[
            pl.BlockSpec((1, _D, _B * _L), lambda i: (i, 0, 0)),
            pl.BlockSpec((_C, _D, _S), lambda i: (0, 0, 0)),
        ],
        out_specs=pl.BlockSpec((_B, 1, _C), lambda i: (i, 0, 0)),
        out_shape=jax.ShapeDtypeStruct((_Q, 1, _C), jnp.float32),
        compiler_params=pltpu.CompilerParams(
            dimension_semantics=("parallel",)),
    )(q, sn)
    return out[:, 0, :]


# speedup vs baseline: 1.5293x; 1.5293x over previous
import jax
import jax.numpy as jnp
from jax.experimental import pallas as pl
from jax.experimental.pallas import tpu as pltpu

_Q = 75
_C = 5
_SHOTS = 5
_D = 384
_L = 196
_S = _SHOTS * _L
_B = 25


def _norm_support_kernel(s_ref, out_ref):
    s = s_ref[0]
    ssq = jnp.sum(s * s, axis=0, keepdims=True)
    out_ref[0] = (s * jax.lax.rsqrt(ssq)).astype(jnp.bfloat16)


def _img2class_kernel(q_ref, sn_ref, out_ref):
    neg = jnp.float32(-jnp.inf)
    qns = []
    for b in range(_B):
        q = q_ref[b]
        qns.append((q * jax.lax.rsqrt(jnp.sum(q * q, axis=0, keepdims=True))).astype(jnp.bfloat16))
    sims = [[None] * _C for _ in range(_B)]
    for c in range(_C):
        sc = sn_ref[c]
        for b in range(_B):
            x = jax.lax.dot_general(
                qns[b], sc, (((0,), (0,)), ((), ())),
                preferred_element_type=jnp.float32,
                precision=jax.lax.Precision.DEFAULT,
            )
            m1 = jnp.max(x, axis=1, keepdims=True)
            x2 = jnp.where(x == m1, neg, x)
            m2 = jnp.max(x2, axis=1, keepdims=True)
            x3 = jnp.where(x2 == m2, neg, x2)
            m3 = jnp.max(x3, axis=1, keepdims=True)
            sims[b][c] = jnp.sum(m1 + m2 + m3, axis=(0, 1), keepdims=True)
    for b in range(_B):
        out_ref[b] = jnp.concatenate(sims[b], axis=1)


def kernel(x1, x2):
    q = x1.reshape(_Q, _D, _L)
    s = x2.reshape(_C, _SHOTS, _D, _L).transpose(0, 2, 1, 3).reshape(_C, _D, _S)
    sn = pl.pallas_call(
        _norm_support_kernel,
        grid=(_C,),
        in_specs=[pl.BlockSpec((1, _D, _S), lambda c: (c, 0, 0))],
        out_specs=pl.BlockSpec((1, _D, _S), lambda c: (c, 0, 0)),
        out_shape=jax.ShapeDtypeStruct((_C, _D, _S), jnp.bfloat16),
    )(s)
    out = pl.pallas_call(
        _img2class_kernel,
        grid=(_Q // _B,),
        in_specs=[
            pl.BlockSpec((_B, _D, _L), lambda i: (i, 0, 0)),
            pl.BlockSpec((_C, _D, _S), lambda i: (0, 0, 0)),
        ],
        out_specs=pl.BlockSpec((_B, 1, _C), lambda i: (i, 0, 0)),
        out_shape=jax.ShapeDtypeStruct((_Q, 1, _C), jnp.float32),
        compiler_params=pltpu.CompilerParams(
            dimension_semantics=("parallel",)),
    )(q, sn)
    return out[:, 0, :]


# speedup vs baseline: 1.6380x; 1.0711x over previous
import jax
import jax.numpy as jnp
from jax.experimental import pallas as pl
from jax.experimental.pallas import tpu as pltpu

_Q = 75
_C = 5
_SHOTS = 5
_D = 384
_L = 196
_S = _SHOTS * _L
_B = 5


def _norm_support_kernel(s_ref, out_ref):
    s = s_ref[0]
    ssq = jnp.sum(s * s, axis=0, keepdims=True)
    out_ref[0] = (s * jax.lax.rsqrt(ssq)).astype(jnp.bfloat16)


def _img2class_kernel(q_ref, sn_ref, out_ref):
    neg = jnp.float32(-jnp.inf)
    qns = []
    for b in range(_B):
        q = q_ref[b]
        qns.append((q * jax.lax.rsqrt(jnp.sum(q * q, axis=0, keepdims=True))).astype(jnp.bfloat16))
    sims = [[None] * _C for _ in range(_B)]
    for c in range(_C):
        sc = sn_ref[c]
        for b in range(_B):
            x = jax.lax.dot_general(
                qns[b], sc, (((0,), (0,)), ((), ())),
                preferred_element_type=jnp.float32,
                precision=jax.lax.Precision.DEFAULT,
            )
            m1 = jnp.max(x, axis=1, keepdims=True)
            tail = jnp.where(x[:, 7 * 128:] == m1, neg, x[:, 7 * 128:])
            a1 = jnp.concatenate(
                [tail, jnp.full((_L, 8 * 128 - _S), neg, jnp.float32)], axis=1)
            a2 = jnp.full((_L, 128), neg, jnp.float32)
            for j in range(7):
                t = jnp.where(x[:, j * 128:(j + 1) * 128] == m1, neg,
                              x[:, j * 128:(j + 1) * 128])
                lo = jnp.minimum(a1, t)
                a1 = jnp.maximum(a1, t)
                a2 = jnp.maximum(a2, lo)
            m2 = jnp.max(a1, axis=1, keepdims=True)
            m3 = jnp.maximum(
                jnp.max(jnp.where(a1 == m2, neg, a1), axis=1, keepdims=True),
                jnp.max(a2, axis=1, keepdims=True))
            sims[b][c] = jnp.sum(m1 + m2 + m3, axis=(0, 1), keepdims=True)
    for b in range(_B):
        out_ref[b] = jnp.concatenate(sims[b], axis=1)


def kernel(x1, x2):
    q = x1.reshape(_Q, _D, _L)
    s = x2.reshape(_C, _SHOTS, _D, _L).transpose(0, 2, 1, 3).reshape(_C, _D, _S)
    sn = pl.pallas_call(
        _norm_support_kernel,
        grid=(_C,),
        in_specs=[pl.BlockSpec((1, _D, _S), lambda c: (c, 0, 0))],
        out_specs=pl.BlockSpec((1, _D, _S), lambda c: (c, 0, 0)),
        out_shape=jax.ShapeDtypeStruct((_C, _D, _S), jnp.bfloat16),
    )(s)
    out = pl.pallas_call(
        _img2class_kernel,
        grid=(_Q // _B,),
        in_specs=[
            pl.BlockSpec((_B, _D, _L), lambda i: (i, 0, 0)),
            pl.BlockSpec((_C, _D, _S), lambda i: (0, 0, 0)),
        ],
        out_specs=pl.BlockSpec((_B, 1, _C), lambda i: (i, 0, 0)),
        out_shape=jax.ShapeDtypeStruct((_Q, 1, _C), jnp.float32),
        compiler_params=pltpu.CompilerParams(
            dimension_semantics=("parallel",)),
    )(q, sn)
    return out[:, 0, :]


# speedup vs baseline: 1.7179x; 1.0488x over previous
import jax
import jax.numpy as jnp
from jax.experimental import pallas as pl
from jax.experimental.pallas import tpu as pltpu

_Q = 75
_C = 5
_SHOTS = 5
_D = 384
_L = 196
_S = _SHOTS * _L
_B = 5


def _norm_support_kernel(s_ref, out_ref):
    s = s_ref[0]
    ssq = jnp.sum(s * s, axis=0, keepdims=True)
    out_ref[0] = (s * jax.lax.rsqrt(ssq)).astype(jnp.bfloat16)


def _img2class_kernel(q_ref, s_ref, out_ref, sn_ref):
    @pl.when(pl.program_id(0) == 0)
    def _normalize_support():
        for c in range(_C):
            s = s_ref[c]
            ssq = jnp.sum(s * s, axis=0, keepdims=True)
            sn_ref[c] = (s * jax.lax.rsqrt(ssq)).astype(jnp.bfloat16)

    neg = jnp.float32(-jnp.inf)
    qns = []
    for b in range(_B):
        q = q_ref[b]
        qns.append((q * jax.lax.rsqrt(jnp.sum(q * q, axis=0, keepdims=True))).astype(jnp.bfloat16))
    sims = [[None] * _C for _ in range(_B)]
    for c in range(_C):
        sc = sn_ref[c]
        for b in range(_B):
            x = jax.lax.dot_general(
                qns[b], sc, (((0,), (0,)), ((), ())),
                preferred_element_type=jnp.float32,
                precision=jax.lax.Precision.DEFAULT,
            )
            m1 = jnp.max(x, axis=1, keepdims=True)
            x2 = jnp.where(x == m1, neg, x)
            m2 = jnp.max(x2, axis=1, keepdims=True)
            x3 = jnp.where(x2 == m2, neg, x2)
            m3 = jnp.max(x3, axis=1, keepdims=True)
            sims[b][c] = jnp.sum(m1 + m2 + m3, axis=(0, 1), keepdims=True)
    for b in range(_B):
        out_ref[b] = jnp.concatenate(sims[b], axis=1)


def kernel(x1, x2):
    q = x1.reshape(_Q, _D, _L)
    s = x2.reshape(_C, _SHOTS, _D, _L).transpose(0, 2, 1, 3).reshape(_C, _D, _S)
    out = pl.pallas_call(
        _img2class_kernel,
        grid=(_Q // _B,),
        in_specs=[
            pl.BlockSpec((_B, _D, _L), lambda i: (i, 0, 0)),
            pl.BlockSpec((_C, _D, _S), lambda i: (0, 0, 0)),
        ],
        out_specs=pl.BlockSpec((_B, 1, _C), lambda i: (i, 0, 0)),
        out_shape=jax.ShapeDtypeStruct((_Q, 1, _C), jnp.float32),
        scratch_shapes=[pltpu.VMEM((_C, _D, _S), jnp.bfloat16)],
        compiler_params=pltpu.CompilerParams(
            dimension_semantics=("arbitrary",)),
    )(q, s)
    return out[:, 0, :]
